# baseline (speedup 1.0000x reference)
import jax
import jax.numpy as jnp
from jax import lax
from jax.experimental import pallas as pl
from jax.experimental.pallas import tpu as pltpu, tpu_sc as plsc

_B = 16
_N = 65536
_NI = 16384
_P = 3
_HW = _NI // 2
_UNROLL = 16


def _gather_body(t_t_hbm, t_idx_hbm, out_hbm, plane_v, idx_v, outv0, outv1, sem, osem):
    wid = lax.axis_index("s") * 2 + lax.axis_index("c")
    b = wid // 2
    half = wid % 2
    hidx = pltpu.async_copy(t_idx_hbm.at[b, pl.ds(half * _HW, _HW)], idx_v, sem)
    hplane = pltpu.async_copy(t_t_hbm.at[0, b], plane_v, sem)
    hidx.wait()
    hplane.wait()
    oh = [None, None]
    for c in range(_P):
        buf = c % 2
        if oh[buf] is not None:
            oh[buf].wait()
        dst = outv0 if buf == 0 else outv1

        def chunk_body(k, carry):
            for u in range(_UNROLL):
                o = (k * _UNROLL + u) * 16
                v = idx_v[pl.ds(o, 16)]
                dst[pl.ds(o, 16)] = plsc.load_gather(plane_v, [v])
            return carry

        lax.fori_loop(0, _HW // (16 * _UNROLL), chunk_body, 0)
        oh[buf] = pltpu.async_copy(
            dst, out_hbm.at[c, b, pl.ds(half * _HW, _HW)], osem
        )
        if c + 1 < _P:
            pltpu.sync_copy(t_t_hbm.at[c + 1, b], plane_v)
    oh[0].wait()
    oh[1].wait()


def kernel(t_in, t_idx):
    b, n, p = t_in.shape
    nidx = t_idx.shape[1]
    t_t = jnp.transpose(t_in, (2, 0, 1))
    idx = t_idx.astype(jnp.int32)
    mesh = plsc.VectorSubcoreMesh(core_axis_name="c", subcore_axis_name="s")
    out = pl.kernel(
        _gather_body,
        out_type=jax.ShapeDtypeStruct((p, b, nidx), jnp.float32),
        mesh=mesh,
        compiler_params=pltpu.CompilerParams(use_tc_tiling_on_sc=True, needs_layout_passes=False, skip_device_barrier=True),
        scratch_types=[
            pltpu.VMEM((_N,), jnp.float32),
            pltpu.VMEM((_HW,), jnp.int32),
            pltpu.VMEM((_HW,), jnp.float32),
            pltpu.VMEM((_HW,), jnp.float32),
            pltpu.SemaphoreType.DMA,
            pltpu.SemaphoreType.DMA,
        ],
    )(t_t, idx)
    return jnp.transpose(out, (1, 2, 0))


# speedup vs baseline: 1.0039x; 1.0039x over previous
import jax
import jax.numpy as jnp
from jax import lax
from jax.experimental import pallas as pl
from jax.experimental.pallas import tpu as pltpu, tpu_sc as plsc

_B = 16
_N = 65536
_NI = 16384
_P = 3
_HW = _NI // 2
_UNROLL = 16


def _gather_body(t_t_hbm, t_idx_hbm, out_hbm, plane_v, idx_v, outv0, outv1, sem, osem):
    wid = lax.axis_index("s") * 2 + lax.axis_index("c")
    b = wid // 2
    half = wid % 2
    hidx = pltpu.async_copy(t_idx_hbm.at[b, pl.ds(half * _HW, _HW)], idx_v, sem)

    def stage(c):
        h0 = pltpu.async_copy(
            t_t_hbm.at[c, b, pl.ds(0, _N // 2)], plane_v.at[pl.ds(0, _N // 2)], sem
        )
        h1 = pltpu.async_copy(
            t_t_hbm.at[c, b, pl.ds(_N // 2, _N // 2)],
            plane_v.at[pl.ds(_N // 2, _N // 2)],
            osem,
        )
        h0.wait()
        h1.wait()

    stage(0)
    hidx.wait()
    oh = [None, None]
    for c in range(_P):
        buf = c % 2
        if oh[buf] is not None:
            oh[buf].wait()
        dst = outv0 if buf == 0 else outv1

        def chunk_body(k, carry):
            for u in range(_UNROLL):
                o = (k * _UNROLL + u) * 16
                v = idx_v[pl.ds(o, 16)]
                dst[pl.ds(o, 16)] = plsc.load_gather(plane_v, [v])
            return carry

        lax.fori_loop(0, _HW // (16 * _UNROLL), chunk_body, 0)
        oh[buf] = pltpu.async_copy(
            dst, out_hbm.at[c, b, pl.ds(half * _HW, _HW)], osem
        )
        if c + 1 < _P:
            stage(c + 1)
    oh[0].wait()
    oh[1].wait()


def kernel(t_in, t_idx):
    b, n, p = t_in.shape
    nidx = t_idx.shape[1]
    t_t = jnp.transpose(t_in, (2, 0, 1))
    idx = t_idx.astype(jnp.int32)
    mesh = plsc.VectorSubcoreMesh(core_axis_name="c", subcore_axis_name="s")
    out = pl.kernel(
        _gather_body,
        out_type=jax.ShapeDtypeStruct((p, b, nidx), jnp.float32),
        mesh=mesh,
        compiler_params=pltpu.CompilerParams(use_tc_tiling_on_sc=True, needs_layout_passes=False),
        scratch_types=[
            pltpu.VMEM((_N,), jnp.float32),
            pltpu.VMEM((_HW,), jnp.int32),
            pltpu.VMEM((_HW,), jnp.float32),
            pltpu.VMEM((_HW,), jnp.float32),
            pltpu.SemaphoreType.DMA,
            pltpu.SemaphoreType.DMA,
        ],
    )(t_t, idx)
    return jnp.transpose(out, (1, 2, 0))


# speedup vs baseline: 1.1433x; 1.1389x over previous
import jax
import jax.numpy as jnp
from jax import lax
from jax.experimental import pallas as pl
from jax.experimental.pallas import tpu as pltpu, tpu_sc as plsc

_B = 16
_N = 65536
_NI = 16384
_P = 3
_HW = _NI // 2
_UNROLL = 16


def _gather_body(t_t_hbm, t_idx_hbm, out_hbm, plane_v, idx_v, outv0, outv1, sem, osem):
    wid = lax.axis_index("s") * 2 + lax.axis_index("c")
    b = wid // 2
    half = wid % 2
    hidx = pltpu.async_copy(t_idx_hbm.at[b, pl.ds(half * _HW, _HW)], idx_v, sem)
    hplane = pltpu.async_copy(t_t_hbm.at[0, b], plane_v, sem)
    hidx.wait()
    hplane.wait()
    oh = [None, None]
    for c in range(_P):
        buf = c % 2
        if oh[buf] is not None:
            oh[buf].wait()
        dst = outv0 if buf == 0 else outv1

        def chunk_body(k, carry):
            vals = []
            for u in range(_UNROLL):
                o = (k * _UNROLL + u) * 16
                v = idx_v[pl.ds(o, 16)]
                vals.append(plsc.load_gather(plane_v, [v]))
            for u in range(_UNROLL):
                o = (k * _UNROLL + u) * 16
                dst[pl.ds(o, 16)] = vals[u]
            return carry

        lax.fori_loop(0, _HW // (16 * _UNROLL), chunk_body, 0)
        oh[buf] = pltpu.async_copy(
            dst, out_hbm.at[c, b, pl.ds(half * _HW, _HW)], osem
        )
        if c + 1 < _P:
            pltpu.sync_copy(t_t_hbm.at[c + 1, b], plane_v)
    oh[0].wait()
    oh[1].wait()


def kernel(t_in, t_idx):
    b, n, p = t_in.shape
    nidx = t_idx.shape[1]
    t_t = jnp.transpose(t_in, (2, 0, 1))
    idx = t_idx.astype(jnp.int32)
    mesh = plsc.VectorSubcoreMesh(core_axis_name="c", subcore_axis_name="s")
    out = pl.kernel(
        _gather_body,
        out_type=jax.ShapeDtypeStruct((p, b, nidx), jnp.float32),
        mesh=mesh,
        compiler_params=pltpu.CompilerParams(use_tc_tiling_on_sc=True, needs_layout_passes=False),
        scratch_types=[
            pltpu.VMEM((_N,), jnp.float32),
            pltpu.VMEM((_HW,), jnp.int32),
            pltpu.VMEM((_HW,), jnp.float32),
            pltpu.VMEM((_HW,), jnp.float32),
            pltpu.SemaphoreType.DMA,
            pltpu.SemaphoreType.DMA,
        ],
    )(t_t, idx)
    return jnp.transpose(out, (1, 2, 0))


# speedup vs baseline: 1.2569x; 1.0993x over previous
import jax
import jax.numpy as jnp
from jax import lax
from jax.experimental import pallas as pl
from jax.experimental.pallas import tpu as pltpu, tpu_sc as plsc

_B = 16
_N = 65536
_NI = 16384
_P = 3
_UNROLL = 16


def _gather_all(idx_v, plane_v, dst):
    def chunk_body(k, carry):
        vals = []
        for u in range(_UNROLL):
            o = (k * _UNROLL + u) * 16
            v = idx_v[pl.ds(o, 16)]
            vals.append(plsc.load_gather(plane_v, [v]))
        for u in range(_UNROLL):
            o = (k * _UNROLL + u) * 16
            dst[pl.ds(o, 16)] = vals[u]
        return carry

    lax.fori_loop(0, _NI // (16 * _UNROLL), chunk_body, 0)


def _gather_body(t_t_hbm, t_idx_hbm, out_hbm, plane_v, idx_v, outv0, outv1,
                 si, sp, so):
    wid = lax.axis_index("s") * 2 + lax.axis_index("c")
    is_a = wid < _B
    b = jnp.where(is_a, wid, wid - _B)
    c0 = jnp.where(is_a, 0, 2)

    hidx = pltpu.async_copy(t_idx_hbm.at[b], idx_v, si)
    hplane = pltpu.async_copy(t_t_hbm.at[c0, b], plane_v, sp)
    hidx.wait()
    hplane.wait()
    _gather_all(idx_v, plane_v, outv0)
    o0 = pltpu.async_copy(outv0, out_hbm.at[c0, b], so)

    @pl.when(is_a)
    def _():
        pltpu.sync_copy(t_t_hbm.at[1, b], plane_v)
        _gather_all(idx_v, plane_v, outv1)
        pltpu.sync_copy(outv1, out_hbm.at[1, b])

    o0.wait()


def kernel(t_in, t_idx):
    b, n, p = t_in.shape
    nidx = t_idx.shape[1]
    t_t = jnp.transpose(t_in, (2, 0, 1))
    idx = t_idx.astype(jnp.int32)
    mesh = plsc.VectorSubcoreMesh(core_axis_name="c", subcore_axis_name="s")
    out = pl.kernel(
        _gather_body,
        out_type=jax.ShapeDtypeStruct((p, b, nidx), jnp.float32),
        mesh=mesh,
        compiler_params=pltpu.CompilerParams(
            use_tc_tiling_on_sc=True, needs_layout_passes=False
        ),
        scratch_types=[
            pltpu.VMEM((_N,), jnp.float32),
            pltpu.VMEM((_NI,), jnp.int32),
            pltpu.VMEM((_NI,), jnp.float32),
            pltpu.VMEM((_NI,), jnp.float32),
            pltpu.SemaphoreType.DMA,
            pltpu.SemaphoreType.DMA,
            pltpu.SemaphoreType.DMA,
        ],
    )(t_t, idx)
    return jnp.transpose(out, (1, 2, 0))
